# baseline (speedup 1.0000x reference)
import functools

import jax
import jax.numpy as jnp
from jax import lax
from jax.experimental import pallas as pl
from jax.experimental.pallas import tpu as pltpu

B = 16384
D = 64
G = 32
CH = B // G


NSEM = 8


def _tc_body(idx_s, table_hbm, out_v, *sems):
    g = pl.program_id(0)

    def issue(c, _):
        k = c * NSEM
        for m in range(NSEM):
            i = idx_s[g * CH + k + m]
            pltpu.make_async_copy(
                table_hbm.at[i], out_v.at[k + m], sems[m]
            ).start()
        return ()

    lax.fori_loop(0, CH // NSEM, issue, ())
    for m in range(NSEM):
        pltpu.make_async_copy(
            table_hbm.at[pl.ds(0, CH // NSEM)],
            out_v.at[pl.ds(0, CH // NSEM)],
            sems[m],
        ).wait()


def kernel(labels, embedding_table):
    idx = labels.astype(jnp.int32)
    grid_spec = pltpu.PrefetchScalarGridSpec(
        num_scalar_prefetch=1,
        grid=(G,),
        in_specs=[pl.BlockSpec(memory_space=pltpu.MemorySpace.HBM)],
        out_specs=pl.BlockSpec((CH, D), lambda g, idx: (g, 0)),
        scratch_shapes=[pltpu.SemaphoreType.DMA] * NSEM,
    )
    return pl.pallas_call(
        _tc_body,
        grid_spec=grid_spec,
        out_shape=jax.ShapeDtypeStruct((B, D), jnp.float32),
    )(idx, embedding_table)


# speedup vs baseline: 1.0582x; 1.0582x over previous
import functools

import jax
import jax.numpy as jnp
from jax import lax
from jax.experimental import pallas as pl
from jax.experimental.pallas import tpu as pltpu
from jax.experimental.pallas import tpu_sc as plsc

B = 16384
D = 64

_info = plsc.get_sparse_core_info()
NC = _info.num_cores
NS = _info.num_subcores
NW = NC * NS

B_SC = 8704
B_TC = B - B_SC
PER_W = B_SC // NW

_mesh = plsc.VectorSubcoreMesh(core_axis_name="c", subcore_axis_name="s")


@functools.partial(
    pl.kernel,
    mesh=_mesh,
    out_type=jax.ShapeDtypeStruct((B_SC, D), jnp.float32),
    scratch_types=[
        pltpu.VMEM((PER_W,), jnp.int32),
        pltpu.VMEM((PER_W, D), jnp.float32),
        pltpu.SemaphoreType.DMA,
        pltpu.SemaphoreType.DMA,
    ],
)
def _embed_sc(table_hbm, idx_hbm, out_hbm, idx_v, rows_v, sem_i, sem):
    wid = lax.axis_index("s") * NC + lax.axis_index("c")
    base = wid * PER_W
    pltpu.async_copy(idx_hbm.at[wid], idx_v, sem_i).wait()

    def body(c, _):
        v = idx_v[pl.ds(c * 16, 16)]
        base_i = c * 16
        for k in range(16):
            pltpu.make_async_copy(
                table_hbm.at[v[k]], rows_v.at[base_i + k], sem
            ).start()
        return ()

    lax.fori_loop(0, PER_W // 16, body, ())
    pltpu.make_async_copy(table_hbm.at[pl.ds(0, PER_W)], rows_v, sem).wait()
    pltpu.sync_copy(rows_v, out_hbm.at[pl.ds(base, PER_W)])


G = 16
CH = B_TC // G
NSEM = 8


def _tc_body(idx_s, table_hbm, out_v, *sems):
    g = pl.program_id(0)

    def issue(c, _):
        k = c * NSEM
        for m in range(NSEM):
            i = idx_s[g * CH + k + m]
            pltpu.make_async_copy(
                table_hbm.at[i], out_v.at[k + m], sems[m]
            ).start()
        return ()

    lax.fori_loop(0, CH // NSEM, issue, ())
    for m in range(NSEM):
        pltpu.make_async_copy(
            table_hbm.at[pl.ds(0, CH // NSEM)],
            out_v.at[pl.ds(0, CH // NSEM)],
            sems[m],
        ).wait()


def _embed_tc(idx, table):
    grid_spec = pltpu.PrefetchScalarGridSpec(
        num_scalar_prefetch=1,
        grid=(G,),
        in_specs=[pl.BlockSpec(memory_space=pltpu.MemorySpace.HBM)],
        out_specs=pl.BlockSpec((CH, D), lambda g, idx: (g, 0)),
        scratch_shapes=[pltpu.SemaphoreType.DMA] * NSEM,
    )
    return pl.pallas_call(
        _tc_body,
        grid_spec=grid_spec,
        out_shape=jax.ShapeDtypeStruct((B_TC, D), jnp.float32),
    )(idx, table)


def kernel(labels, embedding_table):
    idx = labels.astype(jnp.int32)
    out_sc = _embed_sc(embedding_table, idx[:B_SC].reshape(NW, PER_W))
    out_tc = _embed_tc(idx[B_SC:], embedding_table)
    return jnp.concatenate([out_sc, out_tc], axis=0)


# speedup vs baseline: 1.1828x; 1.1178x over previous
import functools

import jax
import jax.numpy as jnp
from jax import lax
from jax.experimental import pallas as pl
from jax.experimental.pallas import tpu as pltpu
from jax.experimental.pallas import tpu_sc as plsc

B = 16384
D = 64

_info = plsc.get_sparse_core_info()
NC = _info.num_cores
NS = _info.num_subcores
NW = NC * NS
B_PER_W = B // NW

_mesh = plsc.VectorSubcoreMesh(core_axis_name="c", subcore_axis_name="s")


@functools.partial(
    pl.kernel,
    mesh=_mesh,
    out_type=jax.ShapeDtypeStruct((B, D), jnp.float32),
    scratch_types=[
        pltpu.VMEM((B_PER_W,), jnp.int32),
        pltpu.VMEM((B_PER_W, D), jnp.float32),
        pltpu.SemaphoreType.DMA,
        pltpu.SemaphoreType.DMA,
    ],
)
def _embed_sc(table_hbm, idx_hbm, out_hbm, idx_v, rows_v, sem_i, sem):
    wid = lax.axis_index("s") * NC + lax.axis_index("c")
    base = wid * B_PER_W
    pltpu.async_copy(idx_hbm.at[wid], idx_v, sem_i).wait()

    def body(c, _):
        v = idx_v[pl.ds(c * 16, 16)]
        base_i = c * 16
        for k in range(16):
            pltpu.make_async_copy(
                table_hbm.at[v[k]], rows_v.at[base_i + k], sem
            ).start()
        return ()

    lax.fori_loop(0, B_PER_W // 16, body, ())
    pltpu.make_async_copy(table_hbm.at[pl.ds(0, B_PER_W)], rows_v, sem).wait()
    pltpu.sync_copy(rows_v, out_hbm.at[pl.ds(base, B_PER_W)])


def kernel(labels, embedding_table):
    idx = labels.astype(jnp.int32).reshape(NW, B_PER_W)
    return _embed_sc(embedding_table, idx)
